# baseline (speedup 1.0000x reference)
import functools

import jax
import jax.numpy as jnp
from jax import lax
from jax.experimental import pallas as pl
from jax.experimental.pallas import tpu as pltpu
from jax.experimental.pallas import tpu_sc as plsc

N = 10000
E = 320000
D = 128
DH = 64
NC = 2
NS = 16
CH = 128
PCH = 64
IB = 8
NBLK = 10
CHUNKS = IB * NBLK
E_PAD = CHUNKS * NC * NS * CH
N_PAD = 10240
ZR = N_PAD // NS
DR = N_PAD // D
TBLK = 1024


def _sc_body(emb_hbm, src_hbm, dst_hbm, w_hbm, acc_out, prm_out,
             acc, deg, hist, sidx, didx, didx2, gbuf, pstage, dbuf, wv, iota,
             gsem, isem):
    c = lax.axis_index("c")
    s = lax.axis_index("s")
    r0 = s * ZR

    zf = jnp.zeros((16,), jnp.float32)

    def zrow(r, carry):
        for k in range(D // 16):
            gbuf[r, pl.ds(k * 16, 16)] = zf
        return carry

    lax.fori_loop(0, CH, zrow, 0)

    def zh(i, carry):
        for k in range(D // 16):
            hist[i, pl.ds(k * 16, 16)] = zf
        return carry

    lax.fori_loop(0, DR, zh, 0)

    for b in range(ZR // CH):
        pltpu.sync_copy(gbuf, acc.at[pl.ds(r0 + b * CH, CH), :])

    @pl.when(s == 0)
    def _():
        pltpu.sync_copy(gbuf.at[pl.ds(0, DR), :], deg)

    it16 = lax.iota(jnp.int32, 16)
    for k in range(DR // 16):
        iota[0, pl.ds(k * 16, 16)] = it16 + 16 * k

    pltpu.sync_copy(w_hbm.at[c], wv)

    plsc.subcore_barrier()

    ones16 = jnp.ones((16,), jnp.float32)

    def hgroup(idxvec):
        plsc.addupdate_scatter(
            hist,
            [lax.shift_right_logical(idxvec, 7),
             lax.bitwise_and(idxvec, D - 1)],
            ones16)

    pltpu.sync_copy(src_hbm.at[c, s, pl.ds(0, IB), :], sidx.at[0])
    pltpu.sync_copy(dst_hbm.at[c, s, pl.ds(0, IB), :], didx.at[0])
    pltpu.sync_copy(dst_hbm.at[1 - c, s, pl.ds(0, IB), :], didx2.at[0])

    def blk(b, carry):
        cur = b & 1
        nxt = 1 - cur
        bn = jnp.minimum(b + 1, NBLK - 1)
        ld0 = pltpu.async_copy(src_hbm.at[c, s, pl.ds(bn * IB, IB), :],
                               sidx.at[nxt], isem[0])
        ld1 = pltpu.async_copy(dst_hbm.at[c, s, pl.ds(bn * IB, IB), :],
                               didx.at[nxt], isem[1])
        ld2 = pltpu.async_copy(dst_hbm.at[1 - c, s, pl.ds(bn * IB, IB), :],
                               didx2.at[nxt], isem[2])
        for j in range(IB):
            g = pltpu.async_copy(emb_hbm.at[sidx.at[cur, j]], gbuf, gsem)
            for k in range(CH // 16):
                hgroup(didx[cur, j, pl.ds(k * 16, 16)])
                hgroup(didx2[cur, j, pl.ds(k * 16, 16)])
            g.wait()
            pltpu.sync_copy(gbuf, acc.at[didx.at[cur, j]], add=True)
        ld0.wait()
        ld1.wait()
        ld2.wait()
        return carry

    lax.fori_loop(0, NBLK, blk, 0)

    pltpu.sync_copy(hist, deg.at[iota.at[0]], add=True)

    plsc.subcore_barrier()

    for b in range(ZR // CH):
        pltpu.sync_copy(acc.at[pl.ds(r0 + b * CH, CH), :], gbuf)
        pltpu.sync_copy(gbuf, acc_out.at[c, pl.ds(r0 + b * CH, CH), :])

    pltpu.sync_copy(deg.at[pl.ds(s * (ZR // D), ZR // D), :], dbuf)
    wvecs = [wv[0, pl.ds(k * 16, 16)] for k in range(DH // 16)]

    for bb in range(ZR // PCH):
        def prow(j, carry, bb=bb):
            row = bb * PCH + j
            dl = plsc.load_gather(
                dbuf, [jnp.full((16,), row // D, jnp.int32),
                       jnp.full((16,), row % D, jnp.int32)])
            for k in range(DH // 16):
                pstage[j, pl.ds(k * 16, 16)] = dl * wvecs[k]
            return carry

        lax.fori_loop(0, PCH, prow, 0)
        pltpu.sync_copy(pstage, prm_out.at[c, pl.ds(r0 + bb * PCH, PCH), :])


_sc_call = pl.kernel(
    _sc_body,
    out_type=(
        jax.ShapeDtypeStruct((NC, N_PAD, D), jnp.float32),
        jax.ShapeDtypeStruct((NC, N_PAD, DH), jnp.float32),
    ),
    mesh=plsc.VectorSubcoreMesh(core_axis_name="c", subcore_axis_name="s"),
    compiler_params=pltpu.CompilerParams(needs_layout_passes=False),
    scratch_types=[
        pltpu.VMEM_SHARED((N_PAD, D), jnp.float32),
        pltpu.VMEM_SHARED((DR, D), jnp.float32),
        pltpu.VMEM((DR, D), jnp.float32),
        pltpu.VMEM((2, IB, CH), jnp.int32),
        pltpu.VMEM((2, IB, CH), jnp.int32),
        pltpu.VMEM((2, IB, CH), jnp.int32),
        pltpu.VMEM((CH, D), jnp.float32),
        pltpu.VMEM((PCH, DH), jnp.float32),
        pltpu.VMEM((ZR // D, D), jnp.float32),
        pltpu.VMEM((1, DH), jnp.float32),
        pltpu.VMEM((1, DR), jnp.int32),
        pltpu.SemaphoreType.DMA,
        (pltpu.SemaphoreType.DMA, pltpu.SemaphoreType.DMA,
         pltpu.SemaphoreType.DMA),
    ],
)


def _tc_body(acc_ref, prm_ref, out_ref):
    out_ref[:, :D] = acc_ref[0] + acc_ref[1]
    out_ref[:, D:D + DH] = prm_ref[0]
    out_ref[:, D + DH:] = prm_ref[1]


_tc_call = pl.pallas_call(
    _tc_body,
    grid=(N_PAD // TBLK,),
    in_specs=[
        pl.BlockSpec((NC, TBLK, D), lambda i: (0, i, 0)),
        pl.BlockSpec((NC, TBLK, DH), lambda i: (0, i, 0)),
    ],
    out_specs=pl.BlockSpec((TBLK, 2 * D), lambda i: (i, 0)),
    out_shape=jax.ShapeDtypeStruct((N_PAD, 2 * D), jnp.float32),
)


@jax.jit
def kernel(graph_embedding, edge_index, weight):
    src = edge_index[0].astype(jnp.int32)
    dst = edge_index[1].astype(jnp.int32)
    pad = E_PAD - E
    src = jnp.concatenate([src, jnp.zeros((pad,), jnp.int32)])
    dst = jnp.concatenate([dst, jnp.full((pad,), N, jnp.int32)])
    srcg = src.reshape(NC, NS, CHUNKS, CH)
    dstg = dst.reshape(NC, NS, CHUNKS, CH)
    w3 = weight.reshape(NC, 1, DH)
    acc_parts, prm_parts = _sc_call(graph_embedding, srcg, dstg, w3)
    return _tc_call(acc_parts, prm_parts)[:N]


# speedup vs baseline: 1.0038x; 1.0038x over previous
import functools

import jax
import jax.numpy as jnp
from jax import lax
from jax.experimental import pallas as pl
from jax.experimental.pallas import tpu as pltpu
from jax.experimental.pallas import tpu_sc as plsc

N = 10000
E = 320000
D = 128
DH = 64
NC = 2
NS = 16
CH = 128
PCH = 32
IB = 8
NBLK = 10
CHUNKS = IB * NBLK
E_PAD = CHUNKS * NC * NS * CH
N_PAD = 10240
ZR = N_PAD // NS
DR = N_PAD // D
TBLK = 1024


def _sc_body(emb_hbm, src_hbm, dst_hbm, acc_out, prm_out,
             acc, deg, hist, sidx, didx, gbuf, pstage, dbuf, iota,
             gsem, isem):
    c = lax.axis_index("c")
    s = lax.axis_index("s")
    r0 = s * ZR

    zf = jnp.zeros((16,), jnp.float32)

    def zrow(r, carry):
        for k in range(D // 16):
            gbuf[r, pl.ds(k * 16, 16)] = zf
        return carry

    lax.fori_loop(0, CH, zrow, 0)

    def zh(i, carry):
        for k in range(D // 16):
            hist[i, pl.ds(k * 16, 16)] = zf
        return carry

    lax.fori_loop(0, DR, zh, 0)

    for b in range(ZR // CH):
        pltpu.sync_copy(gbuf, acc.at[pl.ds(r0 + b * CH, CH), :])

    @pl.when(s == 0)
    def _():
        pltpu.sync_copy(gbuf.at[pl.ds(0, DR), :], deg)

    it16 = lax.iota(jnp.int32, 16)
    for k in range(DR // 16):
        iota[0, pl.ds(k * 16, 16)] = it16 + 16 * k

    plsc.subcore_barrier()

    ones16 = jnp.ones((16,), jnp.float32)

    def hgroup(idxvec):
        plsc.addupdate_scatter(
            hist,
            [lax.shift_right_logical(idxvec, 7),
             lax.bitwise_and(idxvec, D - 1)],
            ones16)

    pltpu.sync_copy(src_hbm.at[c, s, pl.ds(0, IB), :], sidx.at[0])
    pltpu.sync_copy(dst_hbm.at[c, s, pl.ds(0, IB), :], didx.at[0])

    def blk(b, carry):
        cur = b & 1
        nxt = 1 - cur
        bn = jnp.minimum(b + 1, NBLK - 1)
        ld0 = pltpu.async_copy(src_hbm.at[c, s, pl.ds(bn * IB, IB), :],
                               sidx.at[nxt], isem[0])
        ld1 = pltpu.async_copy(dst_hbm.at[c, s, pl.ds(bn * IB, IB), :],
                               didx.at[nxt], isem[1])
        for j in range(IB):
            g = pltpu.async_copy(emb_hbm.at[sidx.at[cur, j]], gbuf, gsem)
            for k in range(CH // 16):
                hgroup(didx[cur, j, pl.ds(k * 16, 16)])
            g.wait()
            pltpu.sync_copy(gbuf, acc.at[didx.at[cur, j]], add=True)
        ld0.wait()
        ld1.wait()
        return carry

    lax.fori_loop(0, NBLK, blk, 0)

    pltpu.sync_copy(hist, deg.at[iota.at[0]], add=True)

    plsc.subcore_barrier()

    for b in range(ZR // CH):
        pltpu.sync_copy(acc.at[pl.ds(r0 + b * CH, CH), :], gbuf)
        pltpu.sync_copy(gbuf, acc_out.at[c, pl.ds(r0 + b * CH, CH), :])

    pltpu.sync_copy(deg.at[pl.ds(s * (ZR // D), ZR // D), :], dbuf)

    for bb in range(ZR // PCH):
        def prow(j, carry, bb=bb):
            row = bb * PCH + j
            dl = plsc.load_gather(
                dbuf, [jnp.full((16,), row // D, jnp.int32),
                       jnp.full((16,), row % D, jnp.int32)])
            for k in range(D // 16):
                pstage[j, pl.ds(k * 16, 16)] = dl
            return carry

        lax.fori_loop(0, PCH, prow, 0)
        pltpu.sync_copy(pstage, prm_out.at[c, pl.ds(r0 + bb * PCH, PCH), :])


_sc_call = pl.kernel(
    _sc_body,
    out_type=(
        jax.ShapeDtypeStruct((NC, N_PAD, D), jnp.float32),
        jax.ShapeDtypeStruct((NC, N_PAD, D), jnp.float32),
    ),
    mesh=plsc.VectorSubcoreMesh(core_axis_name="c", subcore_axis_name="s"),
    compiler_params=pltpu.CompilerParams(needs_layout_passes=False),
    scratch_types=[
        pltpu.VMEM_SHARED((N_PAD, D), jnp.float32),
        pltpu.VMEM_SHARED((DR, D), jnp.float32),
        pltpu.VMEM((DR, D), jnp.float32),
        pltpu.VMEM((2, IB, CH), jnp.int32),
        pltpu.VMEM((2, IB, CH), jnp.int32),
        pltpu.VMEM((CH, D), jnp.float32),
        pltpu.VMEM((PCH, D), jnp.float32),
        pltpu.VMEM((ZR // D, D), jnp.float32),
        pltpu.VMEM((1, DR), jnp.int32),
        pltpu.SemaphoreType.DMA,
        (pltpu.SemaphoreType.DMA, pltpu.SemaphoreType.DMA),
    ],
)


def _tc_body(acc_ref, prm_ref, w_ref, out_ref):
    out_ref[:, :D] = acc_ref[0] + acc_ref[1]
    out_ref[:, D:] = (prm_ref[0] + prm_ref[1]) * w_ref[...]


_tc_call = pl.pallas_call(
    _tc_body,
    grid=(N_PAD // TBLK,),
    in_specs=[
        pl.BlockSpec((NC, TBLK, D), lambda i: (0, i, 0)),
        pl.BlockSpec((NC, TBLK, D), lambda i: (0, i, 0)),
        pl.BlockSpec((1, D), lambda i: (0, 0)),
    ],
    out_specs=pl.BlockSpec((TBLK, 2 * D), lambda i: (i, 0)),
    out_shape=jax.ShapeDtypeStruct((N_PAD, 2 * D), jnp.float32),
)


@jax.jit
def kernel(graph_embedding, edge_index, weight):
    src = edge_index[0].astype(jnp.int32)
    dst = edge_index[1].astype(jnp.int32)
    pad = E_PAD - E
    src = jnp.concatenate([src, jnp.zeros((pad,), jnp.int32)])
    dst = jnp.concatenate([dst, jnp.full((pad,), N, jnp.int32)])
    srcg = src.reshape(NC, NS, CHUNKS, CH)
    dstg = dst.reshape(NC, NS, CHUNKS, CH)
    acc_parts, prm_parts = _sc_call(graph_embedding, srcg, dstg)
    return _tc_call(acc_parts, prm_parts, weight)[:N]


# speedup vs baseline: 1.0051x; 1.0013x over previous
import functools

import jax
import jax.numpy as jnp
from jax import lax
from jax.experimental import pallas as pl
from jax.experimental.pallas import tpu as pltpu
from jax.experimental.pallas import tpu_sc as plsc

N = 10000
E = 320000
D = 128
DH = 64
NC = 2
NS = 16
CH = 128
PCH = 32
IB = 8
NBLK = 10
CHUNKS = IB * NBLK
E_PAD = CHUNKS * NC * NS * CH
N_PAD = 10240
ZR = N_PAD // NS
DR = N_PAD // D
TBLK = 1024


def _sc_body(emb_hbm, src_hbm, dst_hbm, acc_out, prm_out,
             acc, deg, hist, sidx, didx, gbuf, pstage, dbuf, iota,
             gsem, gsem2, isem):
    c = lax.axis_index("c")
    s = lax.axis_index("s")
    r0 = s * ZR

    zf = jnp.zeros((16,), jnp.float32)

    def zrow(r, carry):
        for k in range(D // 16):
            gbuf[r, pl.ds(k * 16, 16)] = zf
        return carry

    lax.fori_loop(0, CH, zrow, 0)

    def zh(i, carry):
        for k in range(D // 16):
            hist[i, pl.ds(k * 16, 16)] = zf
        return carry

    lax.fori_loop(0, DR, zh, 0)

    for b in range(ZR // CH):
        pltpu.sync_copy(gbuf, acc.at[pl.ds(r0 + b * CH, CH), :])

    @pl.when(s == 0)
    def _():
        pltpu.sync_copy(gbuf.at[pl.ds(0, DR), :], deg)

    it16 = lax.iota(jnp.int32, 16)
    for k in range(DR // 16):
        iota[0, pl.ds(k * 16, 16)] = it16 + 16 * k

    plsc.subcore_barrier()

    ones16 = jnp.ones((16,), jnp.float32)

    def hgroup(idxvec):
        plsc.addupdate_scatter(
            hist,
            [lax.shift_right_logical(idxvec, 7),
             lax.bitwise_and(idxvec, D - 1)],
            ones16)

    pltpu.sync_copy(src_hbm.at[c, s, pl.ds(0, IB), :], sidx.at[0])
    pltpu.sync_copy(dst_hbm.at[c, s, pl.ds(0, IB), :], didx.at[0])

    def blk(b, carry):
        cur = b & 1
        nxt = 1 - cur
        bn = jnp.minimum(b + 1, NBLK - 1)
        ld0 = pltpu.async_copy(src_hbm.at[c, s, pl.ds(bn * IB, IB), :],
                               sidx.at[nxt], isem[0])
        ld1 = pltpu.async_copy(dst_hbm.at[c, s, pl.ds(bn * IB, IB), :],
                               didx.at[nxt], isem[1])
        for j in range(IB):
            g0 = pltpu.async_copy(emb_hbm.at[sidx.at[cur, j, pl.ds(0, 64)]],
                                  gbuf.at[pl.ds(0, 64)], gsem)
            g1 = pltpu.async_copy(emb_hbm.at[sidx.at[cur, j, pl.ds(64, 64)]],
                                  gbuf.at[pl.ds(64, 64)], gsem2)
            for k in range(CH // 16):
                hgroup(didx[cur, j, pl.ds(k * 16, 16)])
            g0.wait()
            g1.wait()
            pltpu.sync_copy(gbuf, acc.at[didx.at[cur, j]], add=True)
        ld0.wait()
        ld1.wait()
        return carry

    lax.fori_loop(0, NBLK, blk, 0)

    pltpu.sync_copy(hist, deg.at[iota.at[0]], add=True)

    plsc.subcore_barrier()

    for b in range(ZR // CH):
        pltpu.sync_copy(acc.at[pl.ds(r0 + b * CH, CH), :], gbuf)
        pltpu.sync_copy(gbuf, acc_out.at[c, pl.ds(r0 + b * CH, CH), :])

    pltpu.sync_copy(deg.at[pl.ds(s * (ZR // D), ZR // D), :], dbuf)

    for bb in range(ZR // PCH):
        def prow(j, carry, bb=bb):
            row = bb * PCH + j
            dl = plsc.load_gather(
                dbuf, [jnp.full((16,), row // D, jnp.int32),
                       jnp.full((16,), row % D, jnp.int32)])
            for k in range(D // 16):
                pstage[j, pl.ds(k * 16, 16)] = dl
            return carry

        lax.fori_loop(0, PCH, prow, 0)
        pltpu.sync_copy(pstage, prm_out.at[c, pl.ds(r0 + bb * PCH, PCH), :])


_sc_call = pl.kernel(
    _sc_body,
    out_type=(
        jax.ShapeDtypeStruct((NC, N_PAD, D), jnp.float32),
        jax.ShapeDtypeStruct((NC, N_PAD, D), jnp.float32),
    ),
    mesh=plsc.VectorSubcoreMesh(core_axis_name="c", subcore_axis_name="s"),
    compiler_params=pltpu.CompilerParams(needs_layout_passes=False),
    scratch_types=[
        pltpu.VMEM_SHARED((N_PAD, D), jnp.float32),
        pltpu.VMEM_SHARED((DR, D), jnp.float32),
        pltpu.VMEM((DR, D), jnp.float32),
        pltpu.VMEM((2, IB, CH), jnp.int32),
        pltpu.VMEM((2, IB, CH), jnp.int32),
        pltpu.VMEM((CH, D), jnp.float32),
        pltpu.VMEM((PCH, D), jnp.float32),
        pltpu.VMEM((ZR // D, D), jnp.float32),
        pltpu.VMEM((1, DR), jnp.int32),
        pltpu.SemaphoreType.DMA,
        pltpu.SemaphoreType.DMA,
        (pltpu.SemaphoreType.DMA, pltpu.SemaphoreType.DMA),
    ],
)


def _tc_body(acc_ref, prm_ref, w_ref, out_ref):
    out_ref[:, :D] = acc_ref[0] + acc_ref[1]
    out_ref[:, D:] = (prm_ref[0] + prm_ref[1]) * w_ref[...]


_tc_call = pl.pallas_call(
    _tc_body,
    grid=(N_PAD // TBLK,),
    in_specs=[
        pl.BlockSpec((NC, TBLK, D), lambda i: (0, i, 0)),
        pl.BlockSpec((NC, TBLK, D), lambda i: (0, i, 0)),
        pl.BlockSpec((1, D), lambda i: (0, 0)),
    ],
    out_specs=pl.BlockSpec((TBLK, 2 * D), lambda i: (i, 0)),
    out_shape=jax.ShapeDtypeStruct((N_PAD, 2 * D), jnp.float32),
)


@jax.jit
def kernel(graph_embedding, edge_index, weight):
    src = edge_index[0].astype(jnp.int32)
    dst = edge_index[1].astype(jnp.int32)
    pad = E_PAD - E
    src = jnp.concatenate([src, jnp.zeros((pad,), jnp.int32)])
    dst = jnp.concatenate([dst, jnp.full((pad,), N, jnp.int32)])
    srcg = src.reshape(NC, NS, CHUNKS, CH)
    dstg = dst.reshape(NC, NS, CHUNKS, CH)
    acc_parts, prm_parts = _sc_call(graph_embedding, srcg, dstg)
    return _tc_call(acc_parts, prm_parts, weight)[:N]


# speedup vs baseline: 1.0056x; 1.0004x over previous
import jax
import jax.numpy as jnp
from jax import lax
from jax.experimental import pallas as pl
from jax.experimental.pallas import tpu as pltpu
from jax.experimental.pallas import tpu_sc as plsc

N = 10000
E = 320000
D = 128
NC = 2
NS = 16
CH = 128
PCH = 32
IB = 8
NBLK = 10
CHUNKS = IB * NBLK
E_PAD = CHUNKS * NC * NS * CH
N_PAD = 10240
ZR = N_PAD // NS
DR = N_PAD // D
TBLK = 1024


def _sc_body(emb_hbm, src_hbm, dst_hbm, acc_out, prm_out,
             acc, deg, hist, sidx, didx, gbuf, pstage, dbuf, iota,
             gsem, gsem2, isem):
    c = lax.axis_index("c")
    s = lax.axis_index("s")
    r0 = s * ZR

    zf = jnp.zeros((16,), jnp.float32)

    def zrow(r, carry):
        for k in range(D // 16):
            gbuf[r, pl.ds(k * 16, 16)] = zf
        return carry

    lax.fori_loop(0, CH, zrow, 0)

    def zh(i, carry):
        for k in range(D // 16):
            hist[i, pl.ds(k * 16, 16)] = zf
        return carry

    lax.fori_loop(0, DR, zh, 0)

    for b in range(ZR // CH):
        pltpu.sync_copy(gbuf, acc.at[pl.ds(r0 + b * CH, CH), :])

    @pl.when(s == 0)
    def _():
        pltpu.sync_copy(gbuf.at[pl.ds(0, DR), :], deg)

    it16 = lax.iota(jnp.int32, 16)
    for k in range(DR // 16):
        iota[0, pl.ds(k * 16, 16)] = it16 + 16 * k

    plsc.subcore_barrier()

    ones16 = jnp.ones((16,), jnp.float32)

    def hgroup(idxvec):
        plsc.addupdate_scatter(
            hist,
            [lax.shift_right_logical(idxvec, 7),
             lax.bitwise_and(idxvec, D - 1)],
            ones16)

    pltpu.sync_copy(src_hbm.at[c, s, pl.ds(0, IB), :], sidx.at[0])
    pltpu.sync_copy(dst_hbm.at[c, s, pl.ds(0, IB), :], didx.at[0])

    def blk(b, carry):
        cur = b & 1
        nxt = 1 - cur
        bn = jnp.minimum(b + 1, NBLK - 1)
        ld0 = pltpu.async_copy(src_hbm.at[c, s, pl.ds(bn * IB, IB), :],
                               sidx.at[nxt], isem[0])
        ld1 = pltpu.async_copy(dst_hbm.at[c, s, pl.ds(bn * IB, IB), :],
                               didx.at[nxt], isem[1])
        for j in range(IB):
            g0 = pltpu.async_copy(emb_hbm.at[sidx.at[cur, j, pl.ds(0, 64)]],
                                  gbuf.at[pl.ds(0, 64)], gsem)
            g1 = pltpu.async_copy(emb_hbm.at[sidx.at[cur, j, pl.ds(64, 64)]],
                                  gbuf.at[pl.ds(64, 64)], gsem2)
            for k in range(CH // 16):
                hgroup(didx[cur, j, pl.ds(k * 16, 16)])
            g0.wait()
            g1.wait()
            pltpu.sync_copy(gbuf, acc.at[didx.at[cur, j]], add=True)
        ld0.wait()
        ld1.wait()
        return carry

    lax.fori_loop(0, NBLK, blk, 0)

    pltpu.sync_copy(hist, deg.at[iota.at[0]], add=True)

    plsc.subcore_barrier()

    for b in range(ZR // CH):
        pltpu.sync_copy(acc.at[pl.ds(r0 + b * CH, CH), :], gbuf)
        pltpu.sync_copy(gbuf, acc_out.at[c, pl.ds(r0 + b * CH, CH), :])

    pltpu.sync_copy(deg.at[pl.ds(s * (ZR // D), ZR // D), :], dbuf)

    for bb in range(ZR // PCH):
        def prow(j, carry, bb=bb):
            row = bb * PCH + j
            dl = plsc.load_gather(
                dbuf, [jnp.full((16,), row // D, jnp.int32),
                       jnp.full((16,), row % D, jnp.int32)])
            for k in range(D // 16):
                pstage[j, pl.ds(k * 16, 16)] = dl
            return carry

        lax.fori_loop(0, PCH, prow, 0)
        pltpu.sync_copy(pstage, prm_out.at[c, pl.ds(r0 + bb * PCH, PCH), :])


_sc_call = pl.kernel(
    _sc_body,
    out_type=(
        jax.ShapeDtypeStruct((NC, N_PAD, D), jnp.float32),
        jax.ShapeDtypeStruct((NC, N_PAD, D), jnp.float32),
    ),
    mesh=plsc.VectorSubcoreMesh(core_axis_name="c", subcore_axis_name="s"),
    compiler_params=pltpu.CompilerParams(needs_layout_passes=False),
    scratch_types=[
        pltpu.VMEM_SHARED((N_PAD, D), jnp.float32),
        pltpu.VMEM_SHARED((DR, D), jnp.float32),
        pltpu.VMEM((DR, D), jnp.float32),
        pltpu.VMEM((2, IB, CH), jnp.int32),
        pltpu.VMEM((2, IB, CH), jnp.int32),
        pltpu.VMEM((CH, D), jnp.float32),
        pltpu.VMEM((PCH, D), jnp.float32),
        pltpu.VMEM((ZR // D, D), jnp.float32),
        pltpu.VMEM((1, DR), jnp.int32),
        pltpu.SemaphoreType.DMA,
        pltpu.SemaphoreType.DMA,
        (pltpu.SemaphoreType.DMA, pltpu.SemaphoreType.DMA),
    ],
)


def _tc_body(acc_ref, prm_ref, w_ref, out_ref):
    out_ref[:, :D] = acc_ref[0] + acc_ref[1]
    out_ref[:, D:] = (prm_ref[0] + prm_ref[1]) * w_ref[...]


_tc_call = pl.pallas_call(
    _tc_body,
    grid=(N_PAD // TBLK,),
    in_specs=[
        pl.BlockSpec((NC, TBLK, D), lambda i: (0, i, 0)),
        pl.BlockSpec((NC, TBLK, D), lambda i: (0, i, 0)),
        pl.BlockSpec((1, D), lambda i: (0, 0)),
    ],
    out_specs=pl.BlockSpec((TBLK, 2 * D), lambda i: (i, 0)),
    out_shape=jax.ShapeDtypeStruct((N_PAD, 2 * D), jnp.float32),
)


@jax.jit
def kernel(graph_embedding, edge_index, weight):
    src = edge_index[0].astype(jnp.int32)
    dst = edge_index[1].astype(jnp.int32)
    pad = E_PAD - E
    src = jnp.concatenate([src, jnp.zeros((pad,), jnp.int32)])
    dst = jnp.concatenate([dst, jnp.full((pad,), N, jnp.int32)])
    srcg = src.reshape(NC, NS, CHUNKS, CH)
    dstg = dst.reshape(NC, NS, CHUNKS, CH)
    acc_parts, prm_parts = _sc_call(graph_embedding, srcg, dstg)
    return _tc_call(acc_parts, prm_parts, weight)[:N]
